# Initial kernel scaffold; baseline (speedup 1.0000x reference)
import jax
import jax.numpy as jnp
from jax.experimental import pallas as pl


def kernel(x, gate, w1, w2, w3, sw1, sw2, sw3):
    raise NotImplementedError("write your pallas kernel here")


# speedup vs baseline: 2.2024x; 2.2024x over previous
import functools

import jax
import jax.numpy as jnp
from jax import lax
from jax.experimental import pallas as pl
from jax.experimental.pallas import tpu as pltpu
from jax.experimental.pallas import tpu_sc as plsc

T = 2048
D = 2048
E = 8
F = 1408
K = 2
RB = 128
NBLK = 40
NROWS = NBLK * RB
NC = 2
NS = 16
NW = NC * NS
LANES = 16

_f32 = jnp.float32
_i32 = jnp.int32


def _router_body(x_ref, g_ref, s1_ref, s2_ref, i1_ref, i2_ref):
    x = x_ref[...]
    g = g_ref[...]
    logits = jnp.dot(x, g, preferred_element_type=_f32)
    lane = lax.broadcasted_iota(_i32, logits.shape, 1)
    logits = jnp.where(lane < E, logits, -1e30)
    m = jnp.max(logits, axis=1, keepdims=True)
    p = jnp.exp(logits - m)
    p = jnp.where(lane < E, p, 0.0)
    s = p / jnp.sum(p, axis=1, keepdims=True)
    m1 = jnp.max(s, axis=1, keepdims=True)
    i1 = jnp.min(jnp.where(s >= m1, lane, 128), axis=1, keepdims=True)
    s_x = jnp.where(lane == i1, -1.0, s)
    m2 = jnp.max(s_x, axis=1, keepdims=True)
    i2 = jnp.min(jnp.where(s_x >= m2, lane, 128), axis=1, keepdims=True)
    s1_ref[...] = m1
    s2_ref[...] = m2
    i1_ref[...] = i1
    i2_ref[...] = i2


def _router(xf, gate_padded):
    return pl.pallas_call(
        _router_body,
        out_shape=(
            jax.ShapeDtypeStruct((T, 1), _f32),
            jax.ShapeDtypeStruct((T, 1), _f32),
            jax.ShapeDtypeStruct((T, 1), _i32),
            jax.ShapeDtypeStruct((T, 1), _i32),
        ),
    )(xf, gate_padded)


def _dispatch_body(i1_hbm, i2_hbm, x_hbm, posn_hbm, bexp_hbm, rin_hbm,
                   ids_v, rank_v, posn_v, gidx_v, offs_v, bexp_v,
                   idx_v, rows_v, sem):
    wid = lax.axis_index("s") * NC + lax.axis_index("c")
    iota = lax.iota(_i32, LANES)

    pltpu.sync_copy(i1_hbm, ids_v.at[pl.ds(0, T)])
    pltpu.sync_copy(i2_hbm, ids_v.at[pl.ds(T, T)])

    nvec = (T * K) // LANES

    def body1(p, cnts):
        idsv = ids_v[pl.ds(p * LANES, LANES)]
        rank = jnp.zeros((LANES,), _i32)
        new = []
        for e in range(E):
            msk = idsv == e
            mi = msk.astype(_i32)
            run = jnp.cumsum(mi)
            rank = jnp.where(msk, cnts[e] + run - 1, rank)
            pc = run + lax.rev(jnp.cumsum(lax.rev(mi, (0,))), (0,)) - mi
            new.append(cnts[e] + pc)
        rank_v[pl.ds(p * LANES, LANES)] = rank
        return tuple(new)

    zeros = jnp.zeros((LANES,), _i32)
    cnts = lax.fori_loop(0, nvec, body1, tuple(zeros for _ in range(E)))

    offs_vec = jnp.zeros((LANES,), _i32)
    running = jnp.zeros((LANES,), _i32)
    ends = []
    for e in range(E):
        offs_vec = jnp.where(iota == e, running, offs_vec)
        padded = ((cnts[e] + (RB - 1)) >> 7) << 7
        running = running + padded
        ends.append(running)
    offs_v[...] = offs_vec

    for j in range(3):
        bpos = (iota + j * LANES) * RB
        be = jnp.zeros((LANES,), _i32)
        for e in range(E):
            be = be + (bpos >= ends[e]).astype(_i32)
        bexp_v[pl.ds(j * LANES, LANES)] = jnp.minimum(be, E - 1)

    def bodyz(q, _):
        gidx_v[pl.ds(q * LANES, LANES)] = jnp.zeros((LANES,), _i32)
        return 0
    lax.fori_loop(0, NROWS // LANES, bodyz, 0)

    def body2(p, base):
        idsv = ids_v[pl.ds(p * LANES, LANES)]
        offv = plsc.load_gather(offs_v, [idsv])
        pos = offv + rank_v[pl.ds(p * LANES, LANES)]
        posn_v[pl.ds(p * LANES, LANES)] = pos
        tok = (iota + base) & (T - 1)
        plsc.store_scatter(gidx_v, [pos], tok)
        return base + LANES
    lax.fori_loop(0, nvec, body2, jnp.zeros((LANES,), _i32))

    @pl.when(wid == 0)
    def _():
        pltpu.sync_copy(posn_v, posn_hbm)
        pltpu.sync_copy(bexp_v, bexp_hbm)

    rows_per_w = NROWS // NW
    chunk = 32
    base = wid * rows_per_w

    def bodyg(c, _):
        cb = base + c * chunk
        for q in range(chunk // LANES):
            idx_v[pl.ds(q * LANES, LANES)] = gidx_v[
                pl.ds(cb + q * LANES, LANES)]
        pltpu.async_copy(x_hbm.at[idx_v], rows_v, sem).wait()
        pltpu.sync_copy(rows_v, rin_hbm.at[pl.ds(cb, chunk)])
        return 0
    lax.fori_loop(0, rows_per_w // chunk, bodyg, 0)


def _dispatch(i1f, i2f, xf):
    mesh = plsc.VectorSubcoreMesh(core_axis_name="c", subcore_axis_name="s")
    fn = pl.kernel(
        _dispatch_body,
        mesh=mesh,
        compiler_params=pltpu.CompilerParams(needs_layout_passes=False),
        out_type=(
            jax.ShapeDtypeStruct((T * K,), _i32),
            jax.ShapeDtypeStruct((48,), _i32),
            jax.ShapeDtypeStruct((NROWS, D), _f32),
        ),
        scratch_types=[
            pltpu.VMEM((T * K,), _i32),
            pltpu.VMEM((T * K,), _i32),
            pltpu.VMEM((T * K,), _i32),
            pltpu.VMEM((NROWS,), _i32),
            pltpu.VMEM((LANES,), _i32),
            pltpu.VMEM((48,), _i32),
            pltpu.VMEM((32,), _i32),
            pltpu.VMEM((32, D), _f32),
            pltpu.SemaphoreType.DMA,
        ],
    )
    return fn(i1f, i2f, xf)


def _ffn_body(bexp_ref, rin_ref, w1_ref, w3_ref, w2_ref, out_ref):
    r = rin_ref[...].astype(jnp.bfloat16)
    w1 = w1_ref[0]
    w3 = w3_ref[0]
    w2 = w2_ref[0]
    a = jnp.dot(r, w1, preferred_element_type=_f32)
    b = jnp.dot(r, w3, preferred_element_type=_f32)
    h = a * (1.0 / (1.0 + jnp.exp(-a))) * b
    out_ref[...] = jnp.dot(h.astype(jnp.bfloat16), w2,
                           preferred_element_type=_f32)


def _ffn(bexp, rin, w1, w3, w2):
    grid_spec = pltpu.PrefetchScalarGridSpec(
        num_scalar_prefetch=1,
        grid=(NBLK,),
        in_specs=[
            pl.BlockSpec((RB, D), lambda i, b: (i, 0)),
            pl.BlockSpec((1, D, F), lambda i, b: (b[i], 0, 0)),
            pl.BlockSpec((1, D, F), lambda i, b: (b[i], 0, 0)),
            pl.BlockSpec((1, F, D), lambda i, b: (b[i], 0, 0)),
        ],
        out_specs=pl.BlockSpec((RB, D), lambda i, b: (i, 0)),
    )
    return pl.pallas_call(
        _ffn_body,
        grid_spec=grid_spec,
        out_shape=jax.ShapeDtypeStruct((NROWS, D), _f32),
        compiler_params=pltpu.CompilerParams(
            vmem_limit_bytes=100 * 1024 * 1024),
    )(bexp, rin, w1, w3, w2)


def _cgather_body(rout_hbm, posn_hbm, o1_hbm, o2_hbm, idx_v, rows_v, sem):
    wid = lax.axis_index("s") * NC + lax.axis_index("c")
    per_w = T // NW
    chunk = 32
    base = wid * per_w

    for half, o_hbm in ((0, o1_hbm), (1, o2_hbm)):
        def bodyg(c, _, half=half, o_hbm=o_hbm):
            cb = base + c * chunk
            pltpu.sync_copy(posn_hbm.at[pl.ds(half * T + cb, chunk)], idx_v)
            pltpu.async_copy(rout_hbm.at[idx_v], rows_v, sem).wait()
            pltpu.sync_copy(rows_v, o_hbm.at[pl.ds(cb, chunk)])
            return 0
        lax.fori_loop(0, per_w // chunk, bodyg, 0)


def _cgather(rout, posn):
    mesh = plsc.VectorSubcoreMesh(core_axis_name="c", subcore_axis_name="s")
    fn = pl.kernel(
        _cgather_body,
        mesh=mesh,
        compiler_params=pltpu.CompilerParams(needs_layout_passes=False),
        out_type=(
            jax.ShapeDtypeStruct((T, D), _f32),
            jax.ShapeDtypeStruct((T, D), _f32),
        ),
        scratch_types=[
            pltpu.VMEM((32,), _i32),
            pltpu.VMEM((32, D), _f32),
            pltpu.SemaphoreType.DMA,
        ],
    )
    return fn(rout, posn)


def _combine_body(x_ref, sw1_ref, sw3_ref, sw2_ref, o1_ref, o2_ref,
                  s1_ref, s2_ref, out_ref):
    x = x_ref[...].astype(jnp.bfloat16)
    a = jnp.dot(x, sw1_ref[...], preferred_element_type=_f32)
    b = jnp.dot(x, sw3_ref[...], preferred_element_type=_f32)
    h = a * (1.0 / (1.0 + jnp.exp(-a))) * b
    sh = jnp.dot(h.astype(jnp.bfloat16), sw2_ref[...],
                 preferred_element_type=_f32)
    out_ref[...] = sh + s1_ref[...] * o1_ref[...] + s2_ref[...] * o2_ref[...]


def _combine(xf, sw1, sw3, sw2, o1, o2, s1, s2):
    nb = 16
    rb = T // nb
    return pl.pallas_call(
        _combine_body,
        grid=(nb,),
        in_specs=[
            pl.BlockSpec((rb, D), lambda i: (i, 0)),
            pl.BlockSpec((D, F), lambda i: (0, 0)),
            pl.BlockSpec((D, F), lambda i: (0, 0)),
            pl.BlockSpec((F, D), lambda i: (0, 0)),
            pl.BlockSpec((rb, D), lambda i: (i, 0)),
            pl.BlockSpec((rb, D), lambda i: (i, 0)),
            pl.BlockSpec((rb, 1), lambda i: (i, 0)),
            pl.BlockSpec((rb, 1), lambda i: (i, 0)),
        ],
        out_specs=pl.BlockSpec((rb, D), lambda i: (i, 0)),
        out_shape=jax.ShapeDtypeStruct((T, D), _f32),
        compiler_params=pltpu.CompilerParams(
            vmem_limit_bytes=100 * 1024 * 1024),
    )(xf, sw1, sw3, sw2, o1, o2, s1, s2)


def kernel(x, gate, w1, w2, w3, sw1, sw2, sw3):
    xf = x.reshape(T, D)
    gate_padded = jnp.pad(gate, ((0, 0), (0, 128 - E)))
    s1, s2, i1, i2 = _router(xf, gate_padded)
    posn, bexp, rin = _dispatch(i1.reshape(T), i2.reshape(T), xf)
    bf16 = jnp.bfloat16
    rout = _ffn(bexp, rin, w1.astype(bf16), w3.astype(bf16), w2.astype(bf16))
    o1, o2 = _cgather(rout, posn)
    out = _combine(xf, sw1.astype(bf16), sw3.astype(bf16), sw2.astype(bf16),
                   o1, o2, s1, s2)
    return out.reshape(1, T, D)
